# baseline (speedup 1.0000x reference)
import jax
import jax.numpy as jnp
from jax import lax
from jax.experimental import pallas as pl
from jax.experimental.pallas import tpu as pltpu
from jax.experimental.pallas import tpu_sc as plsc

B = 4096
H = 50
D = 64
VOCAB = 1000000
NC = 2
NS = 16
L = 16
NW = NC * NS
BPW = B // NW
NBLK = VOCAB // 128
TPROWS = VOCAB // 2
NT = 245
CE = 2
CHUNK_IDX = CE * H
NCHUNK = BPW // CE
INV_H = 1.0 / H


TCOLS = 512
TGRID = (VOCAB + TCOLS - 1) // TCOLS
TPR2 = TGRID * 256


def _tr_body(x_ref, o_ref):
  o_ref[:, 0:64] = x_ref[:, 0:256].T
  o_ref[:, 64:128] = x_ref[:, 256:512].T


def _pool_body(tp_hbm, idx_hbm, ne_hbm, out_hbm,
               idx_v, ne_v, rows_v, out_v, gsems):
  wid = lax.axis_index("s") * NC + lax.axis_index("c")
  base = wid * BPW

  pltpu.sync_copy(idx_hbm.at[wid], idx_v)
  pltpu.sync_copy(ne_hbm.at[wid], ne_v)

  def issue(c, u):
    pltpu.async_copy(tp_hbm.at[idx_v.at[c]], rows_v.at[u], gsems.at[u])

  issue(0, 0)
  issue(1, 1)

  def accum_elem(u, el, c):
    r = c * CE + el
    ne = ne_v[r // 8, pl.ds((r % 8) * L, L)][0]
    rbase = el * H

    def lo(j, acc):
      return tuple(acc[k] + rows_v[u, rbase + j, pl.ds(k * L, L)]
                   for k in range(4))

    def hi(j, acc):
      return tuple(acc[k] + rows_v[u, rbase + j, pl.ds(64 + k * L, L)]
                   for k in range(4))

    acc = tuple(jnp.zeros((L,), jnp.float32) for _ in range(4))
    acc = lax.fori_loop(0, ne, lo, acc)
    acc = lax.fori_loop(ne, H, hi, acc)
    for k in range(4):
      out_v[r, pl.ds(k * L, L)] = acc[k] * INV_H

  def step(g, carry):
    for u in range(2):
      c = g * 2 + u
      pltpu.make_async_copy(tp_hbm.at[idx_v.at[c]], rows_v.at[u],
                            gsems.at[u]).wait()
      for el in range(CE):
        accum_elem(u, el, c)

      @pl.when(c + 2 < NCHUNK)
      def _next():
        issue(c + 2, u)
    return carry

  lax.fori_loop(0, NCHUNK // 2, step, 0)
  pltpu.sync_copy(out_v, out_hbm.at[pl.ds(base, BPW)])


@jax.jit
def _run(embt, idx2, ne):
  mesh = plsc.VectorSubcoreMesh(core_axis_name="c", subcore_axis_name="s")
  params = pltpu.CompilerParams(use_tc_tiling_on_sc=True)
  tp = pl.pallas_call(
      _tr_body,
      grid=(TGRID,),
      in_specs=[pl.BlockSpec((D, TCOLS), lambda i: (0, i))],
      out_specs=pl.BlockSpec((TCOLS // 2, 128), lambda i: (i, 0)),
      out_shape=jax.ShapeDtypeStruct((TPR2, 128), jnp.float32),
  )(embt)
  return pl.kernel(
      _pool_body,
      out_type=jax.ShapeDtypeStruct((B, D), jnp.float32),
      mesh=mesh,
      scratch_types=[
          pltpu.VMEM((NCHUNK, CHUNK_IDX), jnp.int32),
          pltpu.VMEM((16, 128), jnp.int32),
          pltpu.VMEM((2, CHUNK_IDX, 128), jnp.float32),
          pltpu.VMEM((BPW, D), jnp.float32),
          pltpu.SemaphoreType.DMA((2,)),
      ],
      compiler_params=params,
  )(tp, idx2, ne)


def kernel(text, text_length, embeddings):
  del text_length
  t = text.astype(jnp.int32)
  pr = (t >> 9) * 256 + (t & 255)
  half = (t >> 8) & 1
  tagged = jnp.sort(half * (1 << 24) + pr, axis=1)
  idx2 = (tagged & ((1 << 24) - 1)).reshape(NW, NCHUNK, CHUNK_IDX)
  ne = H - jnp.sum(half, axis=1, dtype=jnp.int32)
  ne = jnp.repeat(ne.reshape(NW, BPW), 16, axis=1).reshape(NW, 16, 128)
  embt = embeddings.T
  return _run(embt, idx2, ne)


# speedup vs baseline: 2.3805x; 2.3805x over previous
import jax
import jax.numpy as jnp
from jax import lax
from jax.experimental import pallas as pl
from jax.experimental.pallas import tpu as pltpu
from jax.experimental.pallas import tpu_sc as plsc

B = 4096
H = 50
D = 64
VOCAB = 1000000
NC = 2
NS = 16
L = 16
NW = NC * NS
BPW = B // NW
NBLK = VOCAB // 128
TPROWS = VOCAB // 2
NT = 245
CE = 2
CHUNK_IDX = CE * H
NCHUNK = BPW // CE
INV_H = 1.0 / H


TCOLS = 2048
TGRID = (VOCAB + TCOLS - 1) // TCOLS
TPR2 = TGRID * (TCOLS // 2)


def _tr_body(x_ref, o_ref):
  eye = (lax.broadcasted_iota(jnp.int32, (D, D), 0) ==
         lax.broadcasted_iota(jnp.int32, (D, D), 1)).astype(jnp.float32)
  dn = (((0,), (0,)), ((), ()))
  for s in range(TCOLS // 512):
    o_ref[pl.ds(s * 256, 256), 0:64] = lax.dot_general(
        x_ref[:, pl.ds(s * 512, 256)], eye, dn,
        preferred_element_type=jnp.float32)
    o_ref[pl.ds(s * 256, 256), 64:128] = lax.dot_general(
        x_ref[:, pl.ds(s * 512 + 256, 256)], eye, dn,
        preferred_element_type=jnp.float32)


def _pool_body(tp_hbm, idx_hbm, ne_hbm, out_hbm,
               idx_v, ne_v, rows_v, out_v, gsems):
  wid = lax.axis_index("s") * NC + lax.axis_index("c")
  base = wid * BPW

  pltpu.sync_copy(idx_hbm.at[wid], idx_v)
  pltpu.sync_copy(ne_hbm.at[wid], ne_v)

  def issue(c, u):
    pltpu.async_copy(tp_hbm.at[idx_v.at[c]], rows_v.at[u], gsems.at[u])

  issue(0, 0)
  issue(1, 1)

  def accum_elem(u, el, c):
    r = c * CE + el
    ne = ne_v[r // 8, pl.ds((r % 8) * L, L)][0]
    rbase = el * H

    def lo(j, acc):
      return tuple(acc[k] + rows_v[u, rbase + j, pl.ds(k * L, L)]
                   for k in range(4))

    def hi(j, acc):
      return tuple(acc[k] + rows_v[u, rbase + j, pl.ds(64 + k * L, L)]
                   for k in range(4))

    acc = tuple(jnp.zeros((L,), jnp.float32) for _ in range(4))
    acc = lax.fori_loop(0, ne, lo, acc)
    acc = lax.fori_loop(ne, H, hi, acc)
    for k in range(4):
      out_v[r, pl.ds(k * L, L)] = acc[k] * INV_H

  def step(g, carry):
    for u in range(2):
      c = g * 2 + u
      pltpu.make_async_copy(tp_hbm.at[idx_v.at[c]], rows_v.at[u],
                            gsems.at[u]).wait()
      for el in range(CE):
        accum_elem(u, el, c)

      @pl.when(c + 2 < NCHUNK)
      def _next():
        issue(c + 2, u)
    return carry

  lax.fori_loop(0, NCHUNK // 2, step, 0)
  pltpu.sync_copy(out_v, out_hbm.at[pl.ds(base, BPW)])


@jax.jit
def _run(embt, idx2, ne):
  mesh = plsc.VectorSubcoreMesh(core_axis_name="c", subcore_axis_name="s")
  params = pltpu.CompilerParams(use_tc_tiling_on_sc=True)
  tp = pl.pallas_call(
      _tr_body,
      grid=(TGRID,),
      in_specs=[pl.BlockSpec((D, TCOLS), lambda i: (0, i))],
      out_specs=pl.BlockSpec((TCOLS // 2, 128), lambda i: (i, 0)),
      out_shape=jax.ShapeDtypeStruct((TPR2, 128), jnp.float32),
  )(embt)
  return pl.kernel(
      _pool_body,
      out_type=jax.ShapeDtypeStruct((B, D), jnp.float32),
      mesh=mesh,
      scratch_types=[
          pltpu.VMEM((NCHUNK, CHUNK_IDX), jnp.int32),
          pltpu.VMEM((16, 128), jnp.int32),
          pltpu.VMEM((2, CHUNK_IDX, 128), jnp.float32),
          pltpu.VMEM((BPW, D), jnp.float32),
          pltpu.SemaphoreType.DMA((2,)),
      ],
      compiler_params=params,
  )(tp, idx2, ne)


def kernel(text, text_length, embeddings):
  del text_length
  t = text.astype(jnp.int32)
  pr = (t >> 9) * 256 + (t & 255)
  half = (t >> 8) & 1
  tagged = jnp.sort(half * (1 << 24) + pr, axis=1)
  idx2 = (tagged & ((1 << 24) - 1)).reshape(NW, NCHUNK, CHUNK_IDX)
  ne = H - jnp.sum(half, axis=1, dtype=jnp.int32)
  ne = jnp.repeat(ne.reshape(NW, BPW), 16, axis=1).reshape(NW, 16, 128)
  embt = embeddings.T
  return _run(embt, idx2, ne)


# speedup vs baseline: 2.4593x; 1.0331x over previous
import jax
import jax.numpy as jnp
from jax import lax
from jax.experimental import pallas as pl
from jax.experimental.pallas import tpu as pltpu
from jax.experimental.pallas import tpu_sc as plsc

B = 4096
H = 50
D = 64
VOCAB = 1000000
NC = 2
NS = 16
L = 16
NW = NC * NS
BPW = B // NW
NBLK = VOCAB // 128
TPROWS = VOCAB // 2
NT = 245
CE = 2
CHUNK_IDX = CE * H
NCHUNK = BPW // CE
INV_H = 1.0 / H


TCOLS = 2048
TGRID = (VOCAB + TCOLS - 1) // TCOLS
TPR2 = TGRID * (TCOLS // 2)


def _tr_body(x_ref, o_ref):
  eye = (lax.broadcasted_iota(jnp.int32, (256, 256), 0) ==
         lax.broadcasted_iota(jnp.int32, (256, 256), 1)).astype(jnp.bfloat16)
  dn = (((0,), (0,)), ((), ()))
  for h in range(2):
    xcat = jnp.concatenate(
        [x_ref[:, pl.ds(s * 512 + h * 256, 256)] for s in range(4)],
        axis=0).astype(jnp.bfloat16)
    out = lax.dot_general(xcat, eye, dn, preferred_element_type=jnp.float32)
    for s in range(4):
      o_ref[pl.ds(s * 256, 256), pl.ds(h * 64, 64)] = out[:, s * 64:(s + 1) * 64]


def _pool_body(tp_hbm, idx_hbm, ne_hbm, out_hbm,
               idx_v, ne_v, rows_v, out_v, gsems):
  wid = lax.axis_index("s") * NC + lax.axis_index("c")
  base = wid * BPW

  pltpu.sync_copy(idx_hbm.at[wid], idx_v)
  pltpu.sync_copy(ne_hbm.at[wid], ne_v)

  def issue(c, u):
    pltpu.async_copy(tp_hbm.at[idx_v.at[c]], rows_v.at[u], gsems.at[u])

  issue(0, 0)
  issue(1, 1)

  def accum_elem(u, el, c):
    r = c * CE + el
    ne = ne_v[r // 8, pl.ds((r % 8) * L, L)][0]
    rbase = el * H

    def lo(j, acc):
      return tuple(acc[k] + rows_v[u, rbase + j, pl.ds(k * L, L)]
                   for k in range(4))

    def hi(j, acc):
      return tuple(acc[k] + rows_v[u, rbase + j, pl.ds(64 + k * L, L)]
                   for k in range(4))

    acc = tuple(jnp.zeros((L,), jnp.float32) for _ in range(4))
    acc = lax.fori_loop(0, ne, lo, acc)
    acc = lax.fori_loop(ne, H, hi, acc)
    for k in range(4):
      out_v[r, pl.ds(k * L, L)] = acc[k] * INV_H

  def step(g, carry):
    for u in range(2):
      c = g * 2 + u
      pltpu.make_async_copy(tp_hbm.at[idx_v.at[c]], rows_v.at[u],
                            gsems.at[u]).wait()
      for el in range(CE):
        accum_elem(u, el, c)

      @pl.when(c + 2 < NCHUNK)
      def _next():
        issue(c + 2, u)
    return carry

  lax.fori_loop(0, NCHUNK // 2, step, 0)
  pltpu.sync_copy(out_v, out_hbm.at[pl.ds(base, BPW)])


@jax.jit
def _run(embt, idx2, ne):
  mesh = plsc.VectorSubcoreMesh(core_axis_name="c", subcore_axis_name="s")
  params = pltpu.CompilerParams(use_tc_tiling_on_sc=True)
  tp = pl.pallas_call(
      _tr_body,
      grid=(TGRID,),
      in_specs=[pl.BlockSpec((D, TCOLS), lambda i: (0, i))],
      out_specs=pl.BlockSpec((TCOLS // 2, 128), lambda i: (i, 0)),
      out_shape=jax.ShapeDtypeStruct((TPR2, 128), jnp.float32),
  )(embt)
  return pl.kernel(
      _pool_body,
      out_type=jax.ShapeDtypeStruct((B, D), jnp.float32),
      mesh=mesh,
      scratch_types=[
          pltpu.VMEM((NCHUNK, CHUNK_IDX), jnp.int32),
          pltpu.VMEM((16, 128), jnp.int32),
          pltpu.VMEM((2, CHUNK_IDX, 128), jnp.float32),
          pltpu.VMEM((BPW, D), jnp.float32),
          pltpu.SemaphoreType.DMA((2,)),
      ],
      compiler_params=params,
  )(tp, idx2, ne)


def kernel(text, text_length, embeddings):
  del text_length
  t = text.astype(jnp.int32)
  pr = (t >> 9) * 256 + (t & 255)
  half = (t >> 8) & 1
  tagged = jnp.sort(half * (1 << 24) + pr, axis=1)
  idx2 = (tagged & ((1 << 24) - 1)).reshape(NW, NCHUNK, CHUNK_IDX)
  ne = H - jnp.sum(half, axis=1, dtype=jnp.int32)
  ne = jnp.repeat(ne.reshape(NW, BPW), 16, axis=1).reshape(NW, 16, 128)
  embt = embeddings.T
  return _run(embt, idx2, ne)


# speedup vs baseline: 5.3906x; 2.1920x over previous
import jax
import jax.numpy as jnp
from jax import lax
from jax.experimental import pallas as pl
from jax.experimental.pallas import tpu as pltpu
from jax.experimental.pallas import tpu_sc as plsc

B = 4096
H = 50
D = 64
VOCAB = 1000000
NC = 2
NS = 16
L = 16
NW = NC * NS
BPW = B // NW
NBLK = VOCAB // 128
TPROWS = VOCAB // 2
NT = 245
CE = 2
CHUNK_IDX = CE * H
NCHUNK = BPW // CE
INV_H = 1.0 / H


TCOLS = 32768
TGRID = (VOCAB + TCOLS - 1) // TCOLS
TPR2 = TGRID * (TCOLS // 2)


def _tr_body(x_ref, o_ref):
  eye = (lax.broadcasted_iota(jnp.int32, (256, 256), 0) ==
         lax.broadcasted_iota(jnp.int32, (256, 256), 1)).astype(jnp.bfloat16)
  dn = (((0,), (0,)), ((), ()))
  for g in range(TCOLS // 2048):
    outs = []
    for h in range(2):
      xcat = jnp.concatenate(
          [x_ref[:, pl.ds(g * 2048 + s * 512 + h * 256, 256)]
           for s in range(4)],
          axis=0).astype(jnp.bfloat16)
      outs.append(
          lax.dot_general(xcat, eye, dn, preferred_element_type=jnp.float32))
    for s in range(4):
      o_ref[pl.ds(g * 1024 + s * 256, 256), :] = jnp.concatenate(
          [outs[0][:, s * 64:(s + 1) * 64], outs[1][:, s * 64:(s + 1) * 64]],
          axis=1)


def _pool_body(tp_hbm, idx_hbm, ne_hbm, out_hbm,
               idx_v, ne_v, rows_v, out_v, gsems):
  wid = lax.axis_index("s") * NC + lax.axis_index("c")
  base = wid * BPW

  pltpu.sync_copy(idx_hbm.at[wid], idx_v)
  pltpu.sync_copy(ne_hbm.at[wid], ne_v)

  def issue(c, u):
    pltpu.async_copy(tp_hbm.at[idx_v.at[c]], rows_v.at[u], gsems.at[u])

  for c0 in range(4):
    issue(c0, c0)

  def accum_elem(u, el, c):
    r = c * CE + el
    ne = ne_v[r // 8, pl.ds((r % 8) * L, L)][0]
    rbase = el * H

    def lo(j, acc):
      return tuple(acc[k] + rows_v[u, rbase + j, pl.ds(k * L, L)]
                   for k in range(4))

    def hi(j, acc):
      return tuple(acc[k] + rows_v[u, rbase + j, pl.ds(64 + k * L, L)]
                   for k in range(4))

    acc = tuple(jnp.zeros((L,), jnp.float32) for _ in range(4))
    acc = lax.fori_loop(0, ne, lo, acc)
    acc = lax.fori_loop(ne, H, hi, acc)
    for k in range(4):
      out_v[r, pl.ds(k * L, L)] = acc[k] * INV_H

  def step(g, carry):
    for u in range(4):
      c = g * 4 + u
      pltpu.make_async_copy(tp_hbm.at[idx_v.at[c]], rows_v.at[u],
                            gsems.at[u]).wait()
      for el in range(CE):
        accum_elem(u, el, c)

      @pl.when(c + 4 < NCHUNK)
      def _next():
        issue(c + 4, u)
    return carry

  lax.fori_loop(0, NCHUNK // 4, step, 0)
  pltpu.sync_copy(out_v, out_hbm.at[pl.ds(base, BPW)])


@jax.jit
def _run(embt, idx2, ne):
  mesh = plsc.VectorSubcoreMesh(core_axis_name="c", subcore_axis_name="s")
  params = pltpu.CompilerParams(use_tc_tiling_on_sc=True)
  tp = pl.pallas_call(
      _tr_body,
      grid=(TGRID,),
      in_specs=[pl.BlockSpec((D, TCOLS), lambda i: (0, i))],
      out_specs=pl.BlockSpec((TCOLS // 2, 128), lambda i: (i, 0)),
      out_shape=jax.ShapeDtypeStruct((TPR2, 128), jnp.float32),
  )(embt)
  return pl.kernel(
      _pool_body,
      out_type=jax.ShapeDtypeStruct((B, D), jnp.float32),
      mesh=mesh,
      scratch_types=[
          pltpu.VMEM((NCHUNK, CHUNK_IDX), jnp.int32),
          pltpu.VMEM((16, 128), jnp.int32),
          pltpu.VMEM((4, CHUNK_IDX, 128), jnp.float32),
          pltpu.VMEM((BPW, D), jnp.float32),
          pltpu.SemaphoreType.DMA((4,)),
      ],
      compiler_params=params,
  )(tp, idx2, ne)


def kernel(text, text_length, embeddings):
  del text_length
  t = text.astype(jnp.int32)
  pr = (t >> 9) * 256 + (t & 255)
  half = (t >> 8) & 1
  tagged = jnp.sort(half * (1 << 24) + pr, axis=1)
  idx2 = (tagged & ((1 << 24) - 1)).reshape(NW, NCHUNK, CHUNK_IDX)
  ne = H - jnp.sum(half, axis=1, dtype=jnp.int32)
  ne = jnp.repeat(ne.reshape(NW, BPW), 16, axis=1).reshape(NW, 16, 128)
  embt = embeddings.T
  return _run(embt, idx2, ne)
